# baseline (speedup 1.0000x reference)
import jax
import jax.numpy as jnp
from jax.experimental import pallas as pl
from jax.experimental.pallas import tpu as pltpu

_BS = 64


def _body(col_ref, row_ref, o_hbm, pos, sem):
    w, d = col_ref.shape
    h = row_ref.shape[0]
    col = col_ref[...]
    rep = pos.shape[0]
    for r in range(rep):
        for y in range(h):
            pos[r, y * w:(y + 1) * w, 0:d] = col
            pos[r, y * w:(y + 1) * w, d:2 * d] = jnp.broadcast_to(
                row_ref[y:y + 1, :], (w, d))
    copies = [pltpu.make_async_copy(pos, o_hbm.at[pl.ds(b * rep, rep)], sem)
              for b in range(_BS // rep)]
    for c in copies:
        c.start()
    for c in copies:
        c.wait()


def kernel(row_embed, col_embed, bs):
    h, d = row_embed.shape
    w = col_embed.shape[0]
    out = pl.pallas_call(
        _body,
        in_specs=[
            pl.BlockSpec((w, d), lambda: (0, 0)),
            pl.BlockSpec((h, d), lambda: (0, 0)),
        ],
        out_specs=pl.BlockSpec(memory_space=pl.ANY),
        out_shape=jax.ShapeDtypeStruct((_BS, h * w, 2 * d), jnp.float32),
        scratch_shapes=[
            pltpu.VMEM((4, h * w, 2 * d), jnp.float32),
            pltpu.SemaphoreType.DMA,
        ],
    )(col_embed, row_embed)
    return out.reshape(_BS, h, w, 2 * d).transpose(0, 3, 1, 2)
